# baseline (speedup 1.0000x reference)
import functools

import jax
import jax.numpy as jnp
from jax import lax
from jax.experimental import pallas as pl
from jax.experimental.pallas import tpu as pltpu
from jax.experimental.pallas import tpu_sc as plsc

B = 1024
M = 100000
D = 128
H = 8
HD = D // H
TOPN = 10

G1 = 8
N1 = M // G1
BM = 4000
NBLK = M // BM
CH = BM // G1
G2F = 10
CH2 = CH // G2F
N2 = NBLK * CH2
SSEL = 16
TSEL = 20
NCSEL = SSEL * G2F
NCAND = TSEL * G1

NC = 2
NS = 16
NW = NC * NS

_BIG_I = 2**31 - 1


def _extract_topk(vals, ids, k):
    out_v, out_i = [], []
    v = vals
    for _ in range(k):
        mv = jnp.min(v, axis=0, keepdims=True)
        mi = jnp.min(jnp.where(v == mv, ids, _BIG_I), axis=0, keepdims=True)
        out_v.append(mv)
        out_i.append(mi)
        v = jnp.where((v == mv) & (ids == mi), jnp.inf, v)
    return jnp.concatenate(out_v, axis=0), jnp.concatenate(out_i, axis=0)


def _k1_body(e_ref, q_ref, gmin_ref, gmin2_ref):
    e = e_ref[...]
    q = q_ref[...]
    s = lax.dot_general(e, q, (((1,), (1,)), ((), ())),
                        preferred_element_type=jnp.float32)
    e32 = e.astype(jnp.float32)
    m2 = jnp.sum(e32 * e32, axis=1, keepdims=True)
    key = m2 - 2.0 * s
    g1m = jnp.min(key.reshape(CH, G1, B), axis=1)
    gmin_ref[...] = g1m.reshape(1, CH, B)
    gmin2_ref[...] = jnp.min(g1m.reshape(CH2, G2F, B), axis=1).reshape(1, CH2, B)


def _run_k1(e_mem, visit):
    return pl.pallas_call(
        _k1_body,
        grid=(NBLK,),
        in_specs=[
            pl.BlockSpec((BM, D), lambda i: (i, 0)),
            pl.BlockSpec((B, D), lambda i: (0, 0)),
        ],
        out_specs=[
            pl.BlockSpec((1, CH, B), lambda i: (i, 0, 0)),
            pl.BlockSpec((1, CH2, B), lambda i: (i, 0, 0)),
        ],
        out_shape=[
            jax.ShapeDtypeStruct((NBLK, CH, B), jnp.float32),
            jax.ShapeDtypeStruct((NBLK, CH2, B), jnp.float32),
        ],
    )(e_mem, visit)


def _k2a_body(g2_ref, cand_ref, flat_ref):
    g2 = g2_ref[...].reshape(N2, B)
    ids = lax.broadcasted_iota(jnp.int32, (N2, B), 0)
    _, l2win = _extract_topk(g2, ids, SSEL)
    offs = lax.broadcasted_iota(jnp.int32, (SSEL, G2F, B), 1)
    l1cand = l2win.reshape(SSEL, 1, B) * G2F + offs
    lane = lax.broadcasted_iota(jnp.int32, (SSEL, G2F, B), 2)
    cand_ref[...] = l1cand
    flat_ref[...] = l1cand * B + lane


def _run_k2a(gmin2):
    return pl.pallas_call(
        _k2a_body,
        in_specs=[pl.BlockSpec((NBLK, CH2, B), lambda: (0, 0, 0))],
        out_specs=[
            pl.BlockSpec((SSEL, G2F, B), lambda: (0, 0, 0)),
            pl.BlockSpec((SSEL, G2F, B), lambda: (0, 0, 0)),
        ],
        out_shape=[
            jax.ShapeDtypeStruct((SSEL, G2F, B), jnp.int32),
            jax.ShapeDtypeStruct((SSEL, G2F, B), jnp.int32),
        ],
    )(gmin2)


G1E_PER_W = NCSEL * B // NW


def _g1e_body(tbl_hbm, idx_hbm, out_hbm, idx_v, val_v, sem):
    wid = lax.axis_index("s") * NC + lax.axis_index("c")
    base = wid * G1E_PER_W
    pltpu.sync_copy(idx_hbm.at[pl.ds(base, G1E_PER_W)], idx_v)
    pltpu.async_copy(tbl_hbm.at[idx_v], val_v, sem).wait()
    pltpu.sync_copy(val_v, out_hbm.at[pl.ds(base, G1E_PER_W)])


def _run_g1e(gmin_flat, idx_flat):
    mesh = plsc.VectorSubcoreMesh(core_axis_name="c", subcore_axis_name="s")
    return pl.kernel(
        _g1e_body,
        out_type=jax.ShapeDtypeStruct((NCSEL * B,), jnp.float32),
        mesh=mesh,
        scratch_types=[
            pltpu.VMEM((G1E_PER_W,), jnp.int32),
            pltpu.VMEM((G1E_PER_W,), jnp.float32),
            pltpu.SemaphoreType.DMA,
        ],
    )(gmin_flat, idx_flat)


def _k2b_body(v_ref, i_ref, rc_ref):
    v = v_ref[...].reshape(NCSEL, B)
    ids = i_ref[...].reshape(NCSEL, B)
    _, win = _extract_topk(v, ids, TSEL)
    offs = lax.broadcasted_iota(jnp.int32, (TSEL, G1, B), 1)
    rc = win.reshape(TSEL, 1, B) * G1 + offs
    rc_ref[...] = rc.reshape(NCAND, B)


def _run_k2b(cv, ci):
    return pl.pallas_call(
        _k2b_body,
        in_specs=[
            pl.BlockSpec((SSEL, G2F, B), lambda: (0, 0, 0)),
            pl.BlockSpec((SSEL, G2F, B), lambda: (0, 0, 0)),
        ],
        out_specs=pl.BlockSpec((NCAND, B), lambda: (0, 0)),
        out_shape=jax.ShapeDtypeStruct((NCAND, B), jnp.int32),
    )(cv, ci)


G2_PER_W = NCAND * B // NW
G2_CHUNK = 512
G2_NCHUNK = G2_PER_W // G2_CHUNK


def _g2_body(tbl_hbm, idx_hbm, out_hbm, idx_v, rows_v, sem):
    wid = lax.axis_index("s") * NC + lax.axis_index("c")
    base = wid * G2_PER_W
    for c in range(G2_NCHUNK):
        pltpu.sync_copy(idx_hbm.at[pl.ds(base + c * G2_CHUNK, G2_CHUNK)], idx_v)
        pltpu.async_copy(tbl_hbm.at[idx_v], rows_v, sem).wait()
        pltpu.sync_copy(rows_v, out_hbm.at[pl.ds(base + c * G2_CHUNK, G2_CHUNK)])


def _run_g2(e_mem, idx_flat):
    mesh = plsc.VectorSubcoreMesh(core_axis_name="c", subcore_axis_name="s")
    return pl.kernel(
        _g2_body,
        out_type=jax.ShapeDtypeStruct((NCAND * B, D), jnp.float32),
        mesh=mesh,
        scratch_types=[
            pltpu.VMEM((G2_CHUNK,), jnp.int32),
            pltpu.VMEM((G2_CHUNK, D), jnp.float32),
            pltpu.SemaphoreType.DMA,
        ],
    )(e_mem, idx_flat)


QB4 = 128
NQB4 = B // QB4


def _k4_body(r_ref, q_ref, rc_ref, i_ref):
    r = r_ref[...]
    q = q_ref[...]
    key = jnp.sum(r * (r - 2.0 * q.reshape(1, QB4, D)), axis=2)
    _, win = _extract_topk(key, rc_ref[...], TOPN)
    i_ref[...] = win.reshape(1, TOPN, QB4)


def _run_k4(r_cand, visit, rowcand):
    return pl.pallas_call(
        _k4_body,
        grid=(NQB4,),
        in_specs=[
            pl.BlockSpec((NCAND, QB4, D), lambda i: (0, i, 0)),
            pl.BlockSpec((QB4, D), lambda i: (i, 0)),
            pl.BlockSpec((NCAND, QB4), lambda i: (0, i)),
        ],
        out_specs=pl.BlockSpec((1, TOPN, QB4), lambda i: (i, 0, 0)),
        out_shape=jax.ShapeDtypeStruct((NQB4, TOPN, QB4), jnp.int32),
    )(r_cand, visit, rowcand)


G3_PER_W = TOPN * B // NW


def _g3_body(pt_hbm, md_hbm, idx_hbm, outk_hbm, outv_hbm, idx_v, rows_v, sem):
    wid = lax.axis_index("s") * NC + lax.axis_index("c")
    base = wid * G3_PER_W
    pltpu.sync_copy(idx_hbm.at[pl.ds(base, G3_PER_W)], idx_v)
    pltpu.async_copy(pt_hbm.at[idx_v], rows_v, sem).wait()
    pltpu.sync_copy(rows_v, outk_hbm.at[pl.ds(base, G3_PER_W)])
    pltpu.async_copy(md_hbm.at[idx_v], rows_v, sem).wait()
    pltpu.sync_copy(rows_v, outv_hbm.at[pl.ds(base, G3_PER_W)])


def _run_g3(e_pat, e_med, idx_flat):
    mesh = plsc.VectorSubcoreMesh(core_axis_name="c", subcore_axis_name="s")
    return pl.kernel(
        _g3_body,
        out_type=(
            jax.ShapeDtypeStruct((TOPN * B, D), jnp.float32),
            jax.ShapeDtypeStruct((TOPN * B, D), jnp.float32),
        ),
        mesh=mesh,
        scratch_types=[
            pltpu.VMEM((G3_PER_W,), jnp.int32),
            pltpu.VMEM((G3_PER_W, D), jnp.float32),
            pltpu.SemaphoreType.DMA,
        ],
    )(e_pat, e_med, idx_flat)


QB5 = 256
NQB5 = B // QB5


def _ln(x, g, b):
    mu = jnp.mean(x, axis=1, keepdims=True)
    var = jnp.mean((x - mu) * (x - mu), axis=1, keepdims=True)
    return (x - mu) / jnp.sqrt(var + 1e-5) * g + b


def _k5_body(x_ref, kn_ref, vn_ref, wq_ref, wk_ref, wv_ref, bq_ref, bk_ref,
             bv_ref, wo_ref, bo_ref, w1_ref, b1_ref, w2_ref, b2_ref,
             g1_ref, be1_ref, g2_ref, be2_ref, o_ref):
    x = x_ref[...]
    kn = kn_ref[...]
    vn = vn_ref[...]

    def proj(a, w_ref, b_ref):
        return lax.dot_general(a, w_ref[...], (((1,), (1,)), ((), ())),
                               preferred_element_type=jnp.float32) + b_ref[...]

    q_p = proj(x, wq_ref, bq_ref)
    k_p = proj(kn, wk_ref, bk_ref).reshape(QB5, TOPN, D)
    v_p = proj(vn, wv_ref, bv_ref).reshape(QB5, TOPN, D)

    scale = 1.0 / (HD ** 0.5)
    o_heads = []
    for h in range(H):
        qh = q_p[:, h * HD:(h + 1) * HD]
        kh = k_p[:, :, h * HD:(h + 1) * HD]
        vh = v_p[:, :, h * HD:(h + 1) * HD]
        s = jnp.sum(kh * qh.reshape(QB5, 1, HD), axis=2) * scale
        s = s - jnp.max(s, axis=1, keepdims=True)
        e = jnp.exp(s)
        a = e / jnp.sum(e, axis=1, keepdims=True)
        o_heads.append(jnp.sum(vh * a.reshape(QB5, TOPN, 1), axis=1))
    o = jnp.concatenate(o_heads, axis=1)
    o = proj(o, wo_ref, bo_ref)

    x1 = _ln(x + o, g1_ref[...], be1_ref[...])
    hdn = proj(x1, w1_ref, b1_ref)
    hdn = jnp.where(hdn >= 0, hdn, 0.01 * hdn)
    hdn = proj(hdn, w2_ref, b2_ref)
    o_ref[...] = _ln(x1 + hdn, g2_ref[...], be2_ref[...])


def _run_k5(visit, k_nb, v_nb, wq, wk, wv, bq, bk, bv, wo, bo,
            w1, b1, w2, b2, g1, be1, g2, be2):
    full = lambda shape: pl.BlockSpec(shape, lambda i: tuple(0 for _ in shape))
    return pl.pallas_call(
        _k5_body,
        grid=(NQB5,),
        in_specs=[
            pl.BlockSpec((QB5, D), lambda i: (i, 0)),
            pl.BlockSpec((QB5 * TOPN, D), lambda i: (i, 0)),
            pl.BlockSpec((QB5 * TOPN, D), lambda i: (i, 0)),
            full((D, D)), full((D, D)), full((D, D)),
            full((1, D)), full((1, D)), full((1, D)),
            full((D, D)), full((1, D)),
            full((D, D)), full((1, D)), full((D, D)), full((1, D)),
            full((1, D)), full((1, D)), full((1, D)), full((1, D)),
        ],
        out_specs=pl.BlockSpec((QB5, D), lambda i: (i, 0)),
        out_shape=jax.ShapeDtypeStruct((B, D), jnp.float32),
    )(visit, k_nb, v_nb, wq, wk, wv, bq, bk, bv, wo, bo,
      w1, b1, w2, b2, g1, be1, g2, be2)


@jax.jit
def kernel(visit_rep, E_mem_patient_rep, E_mem_med_rep,
           in_proj_weight, in_proj_bias, out_proj_weight, out_proj_bias,
           linear1_weight, linear1_bias, linear2_weight, linear2_bias,
           norm1_weight, norm1_bias, norm2_weight, norm2_bias):
    gmin, gmin2 = _run_k1(E_mem_patient_rep.astype(jnp.bfloat16),
                          visit_rep.astype(jnp.bfloat16))
    l1cand, flatidx = _run_k2a(gmin2)
    cvals = _run_g1e(gmin.reshape(N1 * B), flatidx.reshape(NCSEL * B))
    rowcand = _run_k2b(cvals.reshape(SSEL, G2F, B), l1cand)

    r_cand = _run_g2(E_mem_patient_rep, rowcand.reshape(NCAND * B))
    r_cand = r_cand.reshape(NCAND, B, D)

    i3 = _run_k4(r_cand, visit_rep, rowcand)
    idx_qmaj = jnp.transpose(i3, (0, 2, 1)).reshape(B * TOPN)

    k_nb, v_nb = _run_g3(E_mem_patient_rep, E_mem_med_rep, idx_qmaj)

    r2 = lambda a: a.reshape(1, D)
    return _run_k5(
        visit_rep, k_nb, v_nb,
        in_proj_weight[:D], in_proj_weight[D:2 * D], in_proj_weight[2 * D:],
        r2(in_proj_bias[:D]), r2(in_proj_bias[D:2 * D]), r2(in_proj_bias[2 * D:]),
        out_proj_weight, r2(out_proj_bias),
        linear1_weight, r2(linear1_bias), linear2_weight, r2(linear2_bias),
        r2(norm1_weight), r2(norm1_bias), r2(norm2_weight), r2(norm2_bias),
    )


# speedup vs baseline: 1.6350x; 1.6350x over previous
import functools

import jax
import jax.numpy as jnp
from jax import lax
from jax.experimental import pallas as pl
from jax.experimental.pallas import tpu as pltpu
from jax.experimental.pallas import tpu_sc as plsc

B = 1024
M = 100000
D = 128
H = 8
HD = D // H
TOPN = 10

G1 = 8
N1 = M // G1
BM = 4000
NBLK = M // BM
CH = BM // G1
G2F = 10
CH2 = CH // G2F
N2 = NBLK * CH2
SSEL = 10
TSEL = 10
NCSEL = SSEL * G2F
NCAND = TSEL * G1

NC = 2
NS = 16
NW = NC * NS

_BIG_I = 2**31 - 1


def _extract_topk(vals, ids, k):
    out_v, out_i = [], []
    v = vals
    for _ in range(k):
        mv = jnp.min(v, axis=0, keepdims=True)
        mi = jnp.min(jnp.where(v == mv, ids, _BIG_I), axis=0, keepdims=True)
        out_v.append(mv)
        out_i.append(mi)
        v = jnp.where((v == mv) & (ids == mi), jnp.inf, v)
    return jnp.concatenate(out_v, axis=0), jnp.concatenate(out_i, axis=0)


def _k1_body(e_ref, q_ref, gmin_ref, gmin2_ref):
    e = e_ref[...]
    q = q_ref[...]
    s = lax.dot_general(e, q, (((1,), (1,)), ((), ())),
                        preferred_element_type=jnp.float32)
    m2 = jnp.sum(e * e, axis=1, keepdims=True)
    key = m2 - 2.0 * s
    g1m = jnp.min(key.reshape(CH, G1, B), axis=1)
    gmin_ref[...] = g1m.reshape(1, CH, B)
    gmin2_ref[...] = jnp.min(g1m.reshape(CH2, G2F, B), axis=1).reshape(1, CH2, B)


def _run_k1(e_mem, visit):
    return pl.pallas_call(
        _k1_body,
        grid=(NBLK,),
        in_specs=[
            pl.BlockSpec((BM, D), lambda i: (i, 0)),
            pl.BlockSpec((B, D), lambda i: (0, 0)),
        ],
        out_specs=[
            pl.BlockSpec((1, CH, B), lambda i: (i, 0, 0)),
            pl.BlockSpec((1, CH2, B), lambda i: (i, 0, 0)),
        ],
        out_shape=[
            jax.ShapeDtypeStruct((NBLK, CH, B), jnp.float32),
            jax.ShapeDtypeStruct((NBLK, CH2, B), jnp.float32),
        ],
    )(e_mem, visit)


def _k2a_body(g2_ref, cand_ref, flat_ref):
    g2 = g2_ref[...].reshape(N2, B)
    ids = lax.broadcasted_iota(jnp.int32, (N2, B), 0)
    _, l2win = _extract_topk(g2, ids, SSEL)
    offs = lax.broadcasted_iota(jnp.int32, (SSEL, G2F, B), 1)
    l1cand = l2win.reshape(SSEL, 1, B) * G2F + offs
    lane = lax.broadcasted_iota(jnp.int32, (SSEL, G2F, B), 2)
    cand_ref[...] = l1cand
    flat_ref[...] = l1cand * B + lane


def _run_k2a(gmin2):
    return pl.pallas_call(
        _k2a_body,
        in_specs=[pl.BlockSpec((NBLK, CH2, B), lambda: (0, 0, 0))],
        out_specs=[
            pl.BlockSpec((SSEL, G2F, B), lambda: (0, 0, 0)),
            pl.BlockSpec((SSEL, G2F, B), lambda: (0, 0, 0)),
        ],
        out_shape=[
            jax.ShapeDtypeStruct((SSEL, G2F, B), jnp.int32),
            jax.ShapeDtypeStruct((SSEL, G2F, B), jnp.int32),
        ],
    )(gmin2)


G1E_PER_W = NCSEL * B // NW


def _g1e_body(tbl_hbm, idx_hbm, out_hbm, idx_v, val_v, sem):
    wid = lax.axis_index("s") * NC + lax.axis_index("c")
    base = wid * G1E_PER_W
    pltpu.sync_copy(idx_hbm.at[pl.ds(base, G1E_PER_W)], idx_v)
    pltpu.async_copy(tbl_hbm.at[idx_v], val_v, sem).wait()
    pltpu.sync_copy(val_v, out_hbm.at[pl.ds(base, G1E_PER_W)])


def _run_g1e(gmin_flat, idx_flat):
    mesh = plsc.VectorSubcoreMesh(core_axis_name="c", subcore_axis_name="s")
    return pl.kernel(
        _g1e_body,
        out_type=jax.ShapeDtypeStruct((NCSEL * B,), jnp.float32),
        mesh=mesh,
        scratch_types=[
            pltpu.VMEM((G1E_PER_W,), jnp.int32),
            pltpu.VMEM((G1E_PER_W,), jnp.float32),
            pltpu.SemaphoreType.DMA,
        ],
    )(gmin_flat, idx_flat)


def _k2b_body(v_ref, i_ref, rc_ref):
    v = v_ref[...].reshape(NCSEL, B)
    ids = i_ref[...].reshape(NCSEL, B)
    _, win = _extract_topk(v, ids, TSEL)
    offs = lax.broadcasted_iota(jnp.int32, (TSEL, G1, B), 1)
    rc = win.reshape(TSEL, 1, B) * G1 + offs
    rc_ref[...] = rc.reshape(NCAND, B)


def _run_k2b(cv, ci):
    return pl.pallas_call(
        _k2b_body,
        in_specs=[
            pl.BlockSpec((SSEL, G2F, B), lambda: (0, 0, 0)),
            pl.BlockSpec((SSEL, G2F, B), lambda: (0, 0, 0)),
        ],
        out_specs=pl.BlockSpec((NCAND, B), lambda: (0, 0)),
        out_shape=jax.ShapeDtypeStruct((NCAND, B), jnp.int32),
    )(cv, ci)


G2_PER_W = NCAND * B // NW
G2_CHUNK = 512
G2_NCHUNK = G2_PER_W // G2_CHUNK


def _g2_body(tbl_hbm, idx_hbm, out_hbm, idx_v, rows_v, sem):
    wid = lax.axis_index("s") * NC + lax.axis_index("c")
    base = wid * G2_PER_W
    for c in range(G2_NCHUNK):
        pltpu.sync_copy(idx_hbm.at[pl.ds(base + c * G2_CHUNK, G2_CHUNK)], idx_v)
        pltpu.async_copy(tbl_hbm.at[idx_v], rows_v, sem).wait()
        pltpu.sync_copy(rows_v, out_hbm.at[pl.ds(base + c * G2_CHUNK, G2_CHUNK)])


def _run_g2(e_mem, idx_flat):
    mesh = plsc.VectorSubcoreMesh(core_axis_name="c", subcore_axis_name="s")
    return pl.kernel(
        _g2_body,
        out_type=jax.ShapeDtypeStruct((NCAND * B, D), jnp.float32),
        mesh=mesh,
        scratch_types=[
            pltpu.VMEM((G2_CHUNK,), jnp.int32),
            pltpu.VMEM((G2_CHUNK, D), jnp.float32),
            pltpu.SemaphoreType.DMA,
        ],
    )(e_mem, idx_flat)


QB4 = 128
NQB4 = B // QB4


def _k4_body(r_ref, q_ref, rc_ref, i_ref):
    r = r_ref[...]
    q = q_ref[...]
    key = jnp.sum(r * (r - 2.0 * q.reshape(1, QB4, D)), axis=2)
    _, win = _extract_topk(key, rc_ref[...], TOPN)
    i_ref[...] = win.reshape(1, TOPN, QB4)


def _run_k4(r_cand, visit, rowcand):
    return pl.pallas_call(
        _k4_body,
        grid=(NQB4,),
        in_specs=[
            pl.BlockSpec((NCAND, QB4, D), lambda i: (0, i, 0)),
            pl.BlockSpec((QB4, D), lambda i: (i, 0)),
            pl.BlockSpec((NCAND, QB4), lambda i: (0, i)),
        ],
        out_specs=pl.BlockSpec((1, TOPN, QB4), lambda i: (i, 0, 0)),
        out_shape=jax.ShapeDtypeStruct((NQB4, TOPN, QB4), jnp.int32),
    )(r_cand, visit, rowcand)


G3_PER_W = TOPN * B // NW


def _g3_body(pt_hbm, md_hbm, idx_hbm, outk_hbm, outv_hbm, idx_v, rows_v, sem):
    wid = lax.axis_index("s") * NC + lax.axis_index("c")
    base = wid * G3_PER_W
    pltpu.sync_copy(idx_hbm.at[pl.ds(base, G3_PER_W)], idx_v)
    pltpu.async_copy(pt_hbm.at[idx_v], rows_v, sem).wait()
    pltpu.sync_copy(rows_v, outk_hbm.at[pl.ds(base, G3_PER_W)])
    pltpu.async_copy(md_hbm.at[idx_v], rows_v, sem).wait()
    pltpu.sync_copy(rows_v, outv_hbm.at[pl.ds(base, G3_PER_W)])


def _run_g3(e_pat, e_med, idx_flat):
    mesh = plsc.VectorSubcoreMesh(core_axis_name="c", subcore_axis_name="s")
    return pl.kernel(
        _g3_body,
        out_type=(
            jax.ShapeDtypeStruct((TOPN * B, D), jnp.float32),
            jax.ShapeDtypeStruct((TOPN * B, D), jnp.float32),
        ),
        mesh=mesh,
        scratch_types=[
            pltpu.VMEM((G3_PER_W,), jnp.int32),
            pltpu.VMEM((G3_PER_W, D), jnp.float32),
            pltpu.SemaphoreType.DMA,
        ],
    )(e_pat, e_med, idx_flat)


QB5 = 256
NQB5 = B // QB5


def _ln(x, g, b):
    mu = jnp.mean(x, axis=1, keepdims=True)
    var = jnp.mean((x - mu) * (x - mu), axis=1, keepdims=True)
    return (x - mu) / jnp.sqrt(var + 1e-5) * g + b


def _k5_body(x_ref, kn_ref, vn_ref, wq_ref, wk_ref, wv_ref, bq_ref, bk_ref,
             bv_ref, wo_ref, bo_ref, w1_ref, b1_ref, w2_ref, b2_ref,
             g1_ref, be1_ref, g2_ref, be2_ref, o_ref):
    x = x_ref[...]
    kn = kn_ref[...]
    vn = vn_ref[...]

    def proj(a, w_ref, b_ref):
        return lax.dot_general(a, w_ref[...], (((1,), (1,)), ((), ())),
                               preferred_element_type=jnp.float32) + b_ref[...]

    q_p = proj(x, wq_ref, bq_ref)
    k_p = proj(kn, wk_ref, bk_ref).reshape(QB5, TOPN, D)
    v_p = proj(vn, wv_ref, bv_ref).reshape(QB5, TOPN, D)

    scale = 1.0 / (HD ** 0.5)
    o_heads = []
    for h in range(H):
        qh = q_p[:, h * HD:(h + 1) * HD]
        kh = k_p[:, :, h * HD:(h + 1) * HD]
        vh = v_p[:, :, h * HD:(h + 1) * HD]
        s = jnp.sum(kh * qh.reshape(QB5, 1, HD), axis=2) * scale
        s = s - jnp.max(s, axis=1, keepdims=True)
        e = jnp.exp(s)
        a = e / jnp.sum(e, axis=1, keepdims=True)
        o_heads.append(jnp.sum(vh * a.reshape(QB5, TOPN, 1), axis=1))
    o = jnp.concatenate(o_heads, axis=1)
    o = proj(o, wo_ref, bo_ref)

    x1 = _ln(x + o, g1_ref[...], be1_ref[...])
    hdn = proj(x1, w1_ref, b1_ref)
    hdn = jnp.where(hdn >= 0, hdn, 0.01 * hdn)
    hdn = proj(hdn, w2_ref, b2_ref)
    o_ref[...] = _ln(x1 + hdn, g2_ref[...], be2_ref[...])


def _run_k5(visit, k_nb, v_nb, wq, wk, wv, bq, bk, bv, wo, bo,
            w1, b1, w2, b2, g1, be1, g2, be2):
    full = lambda shape: pl.BlockSpec(shape, lambda i: tuple(0 for _ in shape))
    return pl.pallas_call(
        _k5_body,
        grid=(NQB5,),
        in_specs=[
            pl.BlockSpec((QB5, D), lambda i: (i, 0)),
            pl.BlockSpec((QB5 * TOPN, D), lambda i: (i, 0)),
            pl.BlockSpec((QB5 * TOPN, D), lambda i: (i, 0)),
            full((D, D)), full((D, D)), full((D, D)),
            full((1, D)), full((1, D)), full((1, D)),
            full((D, D)), full((1, D)),
            full((D, D)), full((1, D)), full((D, D)), full((1, D)),
            full((1, D)), full((1, D)), full((1, D)), full((1, D)),
        ],
        out_specs=pl.BlockSpec((QB5, D), lambda i: (i, 0)),
        out_shape=jax.ShapeDtypeStruct((B, D), jnp.float32),
    )(visit, k_nb, v_nb, wq, wk, wv, bq, bk, bv, wo, bo,
      w1, b1, w2, b2, g1, be1, g2, be2)


@jax.jit
def kernel(visit_rep, E_mem_patient_rep, E_mem_med_rep,
           in_proj_weight, in_proj_bias, out_proj_weight, out_proj_bias,
           linear1_weight, linear1_bias, linear2_weight, linear2_bias,
           norm1_weight, norm1_bias, norm2_weight, norm2_bias):
    gmin, gmin2 = _run_k1(E_mem_patient_rep, visit_rep)
    l1cand, flatidx = _run_k2a(gmin2)
    cvals = _run_g1e(gmin.reshape(N1 * B), flatidx.reshape(NCSEL * B))
    rowcand = _run_k2b(cvals.reshape(SSEL, G2F, B), l1cand)

    r_cand = _run_g2(E_mem_patient_rep, rowcand.reshape(NCAND * B))
    r_cand = r_cand.reshape(NCAND, B, D)

    i3 = _run_k4(r_cand, visit_rep, rowcand)
    idx_qmaj = jnp.transpose(i3, (0, 2, 1)).reshape(B * TOPN)

    k_nb, v_nb = _run_g3(E_mem_patient_rep, E_mem_med_rep, idx_qmaj)

    r2 = lambda a: a.reshape(1, D)
    return _run_k5(
        visit_rep, k_nb, v_nb,
        in_proj_weight[:D], in_proj_weight[D:2 * D], in_proj_weight[2 * D:],
        r2(in_proj_bias[:D]), r2(in_proj_bias[D:2 * D]), r2(in_proj_bias[2 * D:]),
        out_proj_weight, r2(out_proj_bias),
        linear1_weight, r2(linear1_bias), linear2_weight, r2(linear2_bias),
        r2(norm1_weight), r2(norm1_bias), r2(norm2_weight), r2(norm2_bias),
    )
